# baseline (speedup 1.0000x reference)
import functools

import jax
import jax.numpy as jnp
from jax.experimental import pallas as pl
from jax.experimental.pallas import tpu as pltpu
from jax.experimental.pallas import tpu_sc as plsc

_PAIR = 2
_BB = 16


def _dot(a, b):
    return jax.lax.dot_general(a, b, (((1,), (1,)), ((), ())),
                               preferred_element_type=jnp.float32)


def _sq_dists(t, i):
    eps2 = jnp.float32(1e-24)
    K = t.shape[0]
    M = i.shape[0]
    ones_row = jnp.ones((1, t.shape[1]), jnp.float32)
    nt2 = _dot(t * t, ones_row)
    ra = jax.lax.rsqrt(jnp.maximum(nt2, eps2))
    a2 = nt2 * ra * ra
    ni2 = _dot(i * i, ones_row)
    rb = jax.lax.rsqrt(jnp.maximum(ni2, eps2))
    b2 = ni2 * rb * rb
    i2 = i * (-2.0 * rb)
    x = _dot(t, i2)
    pa = jnp.concatenate([a2, jnp.ones((K, 1), jnp.float32)], axis=1)
    pb = jnp.concatenate([jnp.ones((M, 1), jnp.float32), b2], axis=1)
    p = _dot(pa, pb)
    return jnp.maximum(x * ra + p, 0.0)


def _insert(q, v):
    a0, a1, a2, a3 = q
    lo = jnp.minimum(a0, v); v = jnp.maximum(a0, v); a0 = lo
    lo = jnp.minimum(a1, v); v = jnp.maximum(a1, v); a1 = lo
    lo = jnp.minimum(a2, v); v = jnp.maximum(a2, v); a2 = lo
    a3 = jnp.minimum(a3, v)
    return (a0, a1, a2, a3)


def _sort_bitonic4(c0, c1, c2, c3):
    lo02 = jnp.minimum(c0, c2); hi02 = jnp.maximum(c0, c2)
    lo13 = jnp.minimum(c1, c3); hi13 = jnp.maximum(c1, c3)
    return (jnp.minimum(lo02, lo13), jnp.maximum(lo02, lo13),
            jnp.minimum(hi02, hi13), jnp.maximum(hi02, hi13))


def _merge4(a, b):
    c0 = jnp.minimum(a[0], b[3])
    c1 = jnp.minimum(a[1], b[2])
    c2 = jnp.minimum(a[2], b[1])
    c3 = jnp.minimum(a[3], b[0])
    return _sort_bitonic4(c0, c1, c2, c3)


def _insert_pair(q, v1, v2):
    w = jnp.minimum(v1, v2)
    z = jnp.maximum(v1, v2)
    c2 = jnp.minimum(q[2], z)
    c3 = jnp.minimum(q[3], w)
    return _sort_bitonic4(q[0], q[1], c2, c3)


def _cand_kernel(t_ref, i_ref, out_ref):
    K = t_ref.shape[1]
    M = i_ref.shape[1]
    W = _PAIR * M
    nchunks = K // 8
    half = nchunks // 2
    big = jnp.float32(3.0e38)

    for p in range(_BB // _PAIR):
        sqs = [_sq_dists(t_ref[_PAIR * p + j], i_ref[_PAIR * p + j])
               for j in range(_PAIR)]
        sq = jnp.concatenate(sqs, axis=1)

        def chunk(c):
            return jax.lax.slice(sq, (8 * c, 0), (8 * c + 8, W))

        def top4_of(chunks):
            w = jnp.minimum(chunk(chunks[0]), chunk(chunks[1]))
            z = jnp.maximum(chunk(chunks[0]), chunk(chunks[1]))
            q = (w, z, jnp.full((8, W), big), jnp.full((8, W), big))
            rest = chunks[2:]
            for k in range(0, len(rest) - 1, 2):
                q = _insert_pair(q, chunk(rest[k]), chunk(rest[k + 1]))
            if len(rest) % 2:
                q = _insert(q, chunk(rest[-1]))
            return q

        qa = top4_of(list(range(0, half)))
        qb = top4_of(list(range(half, nchunks)))
        q = _merge4(qa, qb)

        for s in (4, 2, 1):
            r = tuple(jnp.roll(x, -s, axis=0) for x in q)
            q = _merge4(q, r)

        for j in range(4):
            out_ref[p, j:j + 1, :] = jnp.sqrt(
                jax.lax.slice(q[j], (0, 0), (1, W)))


def _lowest16(a, b):
    return jax.lax.sort(jnp.minimum(a, jax.lax.rev(b, (0,))))


def _sc_topk_kernel(cand_hbm, out_hbm, buf, obuf):
    info = plsc.get_sparse_core_info()
    nc = info.num_cores
    nw = nc * info.num_subcores
    wid = jax.lax.axis_index("s") * nc + jax.lax.axis_index("c")
    npairs = cand_hbm.shape[0]
    per_w = npairs // nw
    lanes_per_batch = cand_hbm.shape[2] // _PAIR
    mask4 = jax.lax.iota(jnp.int32, 16) < 4

    def body(pp, acc):
        p = wid * per_w + pp
        pltpu.sync_copy(cand_hbm.at[p], buf)
        for h in range(_PAIR):
            qs = []
            for j in range(4):
                qs.append([buf[j, pl.ds(h * lanes_per_batch + o * 16, 16)]
                           for o in range(lanes_per_batch // 16)])
            q = (qs[0][0], qs[1][0], qs[2][0], qs[3][0])
            for o in range(1, len(qs[0])):
                q = _merge4(q, (qs[0][o], qs[1][o], qs[2][o], qs[3][o]))
            u = _lowest16(jax.lax.sort(q[0]), jax.lax.sort(q[1]))
            v = _lowest16(jax.lax.sort(q[2]), jax.lax.sort(q[3]))
            w = _lowest16(u, v)
            acc = acc + jnp.where(mask4, w, 0.0)
        return acc

    acc = jax.lax.fori_loop(0, per_w, body, jnp.zeros((16,), jnp.float32))
    obuf[...] = acc
    pltpu.sync_copy(obuf, out_hbm.at[wid])


def _sc_topk(cand):
    npairs = cand.shape[0]
    mesh = plsc.VectorSubcoreMesh(core_axis_name="c", subcore_axis_name="s")
    nw = mesh.num_cores * mesh.num_subcores
    assert npairs % nw == 0
    return pl.kernel(
        _sc_topk_kernel,
        out_type=jax.ShapeDtypeStruct((nw, 16), jnp.float32),
        mesh=mesh,
        compiler_params=pltpu.CompilerParams(needs_layout_passes=False),
        scratch_types=[
            pltpu.VMEM((4, cand.shape[2]), jnp.float32),
            pltpu.VMEM((16,), jnp.float32),
        ],
    )(cand)


_SPLIT = 2


def _cand_stage(tokens, interests):
    B, K, D = tokens.shape
    _, M, _ = interests.shape
    return pl.pallas_call(
        _cand_kernel,
        grid=(B // _BB,),
        in_specs=[
            pl.BlockSpec((_BB, K, D), lambda b: (b, 0, 0)),
            pl.BlockSpec((_BB, M, D), lambda b: (b, 0, 0)),
        ],
        out_specs=pl.BlockSpec((_BB // _PAIR, 4, _PAIR * M),
                               lambda b: (b, 0, 0)),
        out_shape=jax.ShapeDtypeStruct((B // _PAIR, 4, _PAIR * M),
                                       jnp.float32),
    )(tokens, interests)


def kernel(tokens, interests):
    B = tokens.shape[0]
    h = B // _SPLIT
    partials = [
        _sc_topk(_cand_stage(tokens[s * h:(s + 1) * h],
                             interests[s * h:(s + 1) * h]))
        for s in range(_SPLIT)
    ]
    total = sum(jnp.sum(p) for p in partials)
    return (total / (B * 4)).astype(jnp.float32)


# speedup vs baseline: 1.6981x; 1.6981x over previous
import functools

import jax
import jax.numpy as jnp
from jax.experimental import pallas as pl
from jax.experimental.pallas import tpu as pltpu
from jax.experimental.pallas import tpu_sc as plsc

_PAIR = 2
_BB = 16


def _dot(a, b):
    return jax.lax.dot_general(a, b, (((1,), (1,)), ((), ())),
                               preferred_element_type=jnp.float32)


def _sq_dists(t, i):
    eps2 = jnp.float32(1e-24)
    K = t.shape[0]
    M = i.shape[0]
    ones_row = jnp.ones((1, t.shape[1]), jnp.float32)
    nt2 = _dot(t * t, ones_row)
    ra = jax.lax.rsqrt(jnp.maximum(nt2, eps2))
    a2 = nt2 * ra * ra
    ni2 = _dot(i * i, ones_row)
    rb = jax.lax.rsqrt(jnp.maximum(ni2, eps2))
    b2 = ni2 * rb * rb
    i2 = i * (-2.0 * rb)
    x = _dot(t, i2)
    pa = jnp.concatenate([a2, jnp.ones((K, 1), jnp.float32)], axis=1)
    pb = jnp.concatenate([jnp.ones((M, 1), jnp.float32), b2], axis=1)
    p = _dot(pa, pb)
    return jnp.maximum(x * ra + p, 0.0)


def _insert(q, v):
    a0, a1, a2, a3 = q
    lo = jnp.minimum(a0, v); v = jnp.maximum(a0, v); a0 = lo
    lo = jnp.minimum(a1, v); v = jnp.maximum(a1, v); a1 = lo
    lo = jnp.minimum(a2, v); v = jnp.maximum(a2, v); a2 = lo
    a3 = jnp.minimum(a3, v)
    return (a0, a1, a2, a3)


def _sort_bitonic4(c0, c1, c2, c3):
    lo02 = jnp.minimum(c0, c2); hi02 = jnp.maximum(c0, c2)
    lo13 = jnp.minimum(c1, c3); hi13 = jnp.maximum(c1, c3)
    return (jnp.minimum(lo02, lo13), jnp.maximum(lo02, lo13),
            jnp.minimum(hi02, hi13), jnp.maximum(hi02, hi13))


def _merge4(a, b):
    c0 = jnp.minimum(a[0], b[3])
    c1 = jnp.minimum(a[1], b[2])
    c2 = jnp.minimum(a[2], b[1])
    c3 = jnp.minimum(a[3], b[0])
    return _sort_bitonic4(c0, c1, c2, c3)


def _insert_pair(q, v1, v2):
    w = jnp.minimum(v1, v2)
    z = jnp.maximum(v1, v2)
    c2 = jnp.minimum(q[2], z)
    c3 = jnp.minimum(q[3], w)
    return _sort_bitonic4(q[0], q[1], c2, c3)


def _cand_kernel(t_ref, i_ref, out_ref):
    K = t_ref.shape[1]
    M = i_ref.shape[1]
    W = _PAIR * M
    nchunks = K // 8
    half = nchunks // 2
    big = jnp.float32(3.0e38)

    for p in range(_BB // _PAIR):
        sqs = [_sq_dists(t_ref[_PAIR * p + j], i_ref[_PAIR * p + j])
               for j in range(_PAIR)]
        sq = jnp.concatenate(sqs, axis=1)

        def chunk(c):
            return jax.lax.slice(sq, (8 * c, 0), (8 * c + 8, W))

        def top4_of(chunks):
            w = jnp.minimum(chunk(chunks[0]), chunk(chunks[1]))
            z = jnp.maximum(chunk(chunks[0]), chunk(chunks[1]))
            q = (w, z, jnp.full((8, W), big), jnp.full((8, W), big))
            rest = chunks[2:]
            for k in range(0, len(rest) - 1, 2):
                q = _insert_pair(q, chunk(rest[k]), chunk(rest[k + 1]))
            if len(rest) % 2:
                q = _insert(q, chunk(rest[-1]))
            return q

        qa = top4_of(list(range(0, half)))
        qb = top4_of(list(range(half, nchunks)))
        q = _merge4(qa, qb)

        for s in (4, 2, 1):
            r = tuple(jnp.roll(x, -s, axis=0) for x in q)
            q = _merge4(q, r)

        for j in range(4):
            out_ref[p, j:j + 1, :] = jnp.sqrt(
                jax.lax.slice(q[j], (0, 0), (1, W)))


def _lowest16(a, b):
    return jax.lax.sort(jnp.minimum(a, jax.lax.rev(b, (0,))))


def _sc_topk_kernel(cand_hbm, out_hbm, buf, obuf):
    info = plsc.get_sparse_core_info()
    nc = info.num_cores
    nw = nc * info.num_subcores
    wid = jax.lax.axis_index("s") * nc + jax.lax.axis_index("c")
    npairs = cand_hbm.shape[0]
    per_w = npairs // nw
    lanes_per_batch = cand_hbm.shape[2] // _PAIR
    mask4 = jax.lax.iota(jnp.int32, 16) < 4

    def body(pp, acc):
        p = wid * per_w + pp
        pltpu.sync_copy(cand_hbm.at[p], buf)
        for h in range(_PAIR):
            qs = []
            for j in range(4):
                qs.append([buf[j, pl.ds(h * lanes_per_batch + o * 16, 16)]
                           for o in range(lanes_per_batch // 16)])
            q = (qs[0][0], qs[1][0], qs[2][0], qs[3][0])
            for o in range(1, len(qs[0])):
                q = _merge4(q, (qs[0][o], qs[1][o], qs[2][o], qs[3][o]))
            u = _lowest16(jax.lax.sort(q[0]), jax.lax.sort(q[1]))
            v = _lowest16(jax.lax.sort(q[2]), jax.lax.sort(q[3]))
            w = _lowest16(u, v)
            acc = acc + jnp.where(mask4, w, 0.0)
        return acc

    acc = jax.lax.fori_loop(0, per_w, body, jnp.zeros((16,), jnp.float32))
    obuf[...] = acc
    pltpu.sync_copy(obuf, out_hbm.at[wid])


def _sc_topk(cand):
    npairs = cand.shape[0]
    mesh = plsc.VectorSubcoreMesh(core_axis_name="c", subcore_axis_name="s")
    nw = mesh.num_cores * mesh.num_subcores
    assert npairs % nw == 0
    return pl.kernel(
        _sc_topk_kernel,
        out_type=jax.ShapeDtypeStruct((nw, 16), jnp.float32),
        mesh=mesh,
        compiler_params=pltpu.CompilerParams(needs_layout_passes=False),
        scratch_types=[
            pltpu.VMEM((4, cand.shape[2]), jnp.float32),
            pltpu.VMEM((16,), jnp.float32),
        ],
    )(cand)


def _cand_stage(tokens, interests):
    B, K, D = tokens.shape
    _, M, _ = interests.shape
    return pl.pallas_call(
        _cand_kernel,
        grid=(B // _BB,),
        in_specs=[
            pl.BlockSpec((_BB, K, D), lambda b: (b, 0, 0)),
            pl.BlockSpec((_BB, M, D), lambda b: (b, 0, 0)),
        ],
        out_specs=pl.BlockSpec((_BB // _PAIR, 4, _PAIR * M),
                               lambda b: (b, 0, 0)),
        out_shape=jax.ShapeDtypeStruct((B // _PAIR, 4, _PAIR * M),
                                       jnp.float32),
    )(tokens, interests)


def kernel(tokens, interests):
    B = tokens.shape[0]
    partials = _sc_topk(_cand_stage(tokens, interests))
    return (jnp.sum(partials) / (B * 4)).astype(jnp.float32)
